# baseline (speedup 1.0000x reference)
import functools

import jax
import jax.numpy as jnp
from jax import lax
from jax.experimental import pallas as pl
from jax.experimental.pallas import tpu as pltpu
from jax.experimental.pallas import tpu_sc as plsc

D = 1024
D2 = D // 2
NTOK = 8192
NWORK = 32
TPW = NTOK // NWORK
C = 8
NCHUNK = TPW // C
NU = D2 // 16
UNROLL = 16
EPS = 1e-12
_PACK = plsc.PackFormat.INTERLEAVED


def _rsqrt16(v):
    i = lax.bitcast_convert_type(v, jnp.int32)
    i = jnp.int32(0x5F3759DF) - (i >> 1)
    y = lax.bitcast_convert_type(i, jnp.float32)
    for _ in range(3):
        y = y * (1.5 - 0.5 * v * y * y)
    return y


def _lane_total(v):
    lanes = lax.iota(jnp.int32, 16)
    for k in (8, 4, 2, 1):
        v = v + v.at[lanes ^ k].get(mode="promise_in_bounds")
    return v


def _bf(x):
    return plsc.bitcast(x, jnp.bfloat16)


def _i32(x):
    return plsc.bitcast(x, jnp.int32)


def _sc_body(idx_hbm, table_hbm, gamma_hbm, beta_hbm, out_hbm,
             idx_v, rows_v, buf_v, obuf_v, gam_v, bet_v,
             gsem0, gsem1, osem0, osem1):
    wid = lax.axis_index("s") * 2 + lax.axis_index("c")
    tok0 = wid * TPW

    pltpu.sync_copy(gamma_hbm, gam_v)
    pltpu.sync_copy(beta_hbm, bet_v)
    pltpu.sync_copy(idx_hbm.at[pl.ds(wid * NCHUNK, NCHUNK)], idx_v)

    def issue_gather(g, rows, gsem):
        return pltpu.async_copy(table_hbm.at[idx_v.at[g]], rows, gsem)

    def wait_gather(rows, gsem):
        pltpu.make_async_copy(table_hbm.at[idx_v.at[0]], rows, gsem).wait()

    def compute_chunk(g, rows, buf, obuf, osem):
        base_tok = tok0 + g * C

        def d_sum(d8, carry):
            s, q = carry
            ns, nq = [], []
            for t in range(C):
                st, qt = s[t], q[t]
                for j in range(UNROLL):
                    off = pl.ds(d8 * (16 * UNROLL) + j * 16, 16)
                    v = ((_bf(rows[4 * t + 0, off]) + _bf(rows[4 * t + 1, off]))
                         + (_bf(rows[4 * t + 2, off]) + _bf(rows[4 * t + 3, off])))
                    buf[t, off] = _i32(v)
                    a, b = plsc.unpack(v, format=_PACK)
                    st = st + (a + b)
                    qt = qt + (a * a + b * b)
                ns.append(st)
                nq.append(qt)
            return (tuple(ns), tuple(nq))

        zeros = jnp.zeros((16,), jnp.float32)
        s, q = lax.fori_loop(0, NU // UNROLL, d_sum,
                             ((zeros,) * C, (zeros,) * C))

        mvecs, rstds = [], []
        for t in range(C):
            mvec = _lane_total(s[t]) * (1.0 / D)
            var = _lane_total(q[t]) * (1.0 / D) - mvec * mvec
            rstd = _rsqrt16(var + EPS)
            mvecs.append(plsc.pack(mvec, mvec, format=_PACK))
            rstds.append(plsc.pack(rstd, rstd, format=_PACK))

        def d_norm(d8, carry):
            for j in range(UNROLL):
                o = d8 * (16 * UNROLL) + j * 16
                off = pl.ds(o, 16)
                gv = _bf(gam_v[off])
                bv = _bf(bet_v[off])
                for t in range(C):
                    v = _bf(buf[t, off])
                    y = (v - mvecs[t]) * rstds[t] * gv + bv
                    ylo, yhi = plsc.unpack(y, format=_PACK)
                    obuf[t, off] = ylo
                    obuf[t, pl.ds(D2 + o, 16)] = yhi
            return carry

        lax.fori_loop(0, NU // UNROLL, d_norm, 0)
        return pltpu.async_copy(obuf, out_hbm.at[pl.ds(base_tok, C)], osem)

    def wait_out(obuf, osem):
        pltpu.make_async_copy(obuf, out_hbm.at[pl.ds(tok0, C)], osem).wait()

    issue_gather(0, rows_v.at[0], gsem0)

    def pipe_body(g2, _):
        g = 2 * g2
        issue_gather(g + 1, rows_v.at[1], gsem1)
        wait_gather(rows_v.at[0], gsem0)

        @pl.when(g2 >= 1)
        def _():
            wait_out(obuf_v.at[0], osem0)

        compute_chunk(g, rows_v.at[0], buf_v.at[0], obuf_v.at[0], osem0)

        @pl.when(g + 2 < NCHUNK)
        def _():
            issue_gather(g + 2, rows_v.at[0], gsem0)

        wait_gather(rows_v.at[1], gsem1)

        @pl.when(g2 >= 1)
        def _():
            wait_out(obuf_v.at[1], osem1)

        compute_chunk(g + 1, rows_v.at[1], buf_v.at[1], obuf_v.at[1], osem1)
        return 0

    lax.fori_loop(0, NCHUNK // 2, pipe_body, 0)
    wait_out(obuf_v.at[0], osem0)
    wait_out(obuf_v.at[1], osem1)


@jax.jit
def _sc_call(idx, table, gamma, beta):
    mesh = plsc.VectorSubcoreMesh(core_axis_name="c", subcore_axis_name="s")
    kfn = functools.partial(
        pl.kernel, mesh=mesh,
        compiler_params=pltpu.CompilerParams(needs_layout_passes=False),
        out_type=jax.ShapeDtypeStruct((NTOK, D), jnp.float32),
        scratch_types=[
            pltpu.VMEM((NCHUNK, 4 * C), jnp.int32),
            pltpu.VMEM((2, 4 * C, D2), jnp.int32),
            pltpu.VMEM((2, C, D2), jnp.int32),
            pltpu.VMEM((2, C, D), jnp.float32),
            pltpu.VMEM((D2,), jnp.int32),
            pltpu.VMEM((D2,), jnp.int32),
            pltpu.SemaphoreType.DMA,
            pltpu.SemaphoreType.DMA,
            pltpu.SemaphoreType.DMA,
            pltpu.SemaphoreType.DMA,
        ],
    )(_sc_body)
    return kfn(idx, table, gamma, beta)


def _as_i32_pairs(x_f32):
    bits = lax.bitcast_convert_type(x_f32, jnp.uint32)
    r = bits + jnp.uint32(0x7FFF) + ((bits >> 16) & jnp.uint32(1))
    bf = r >> 16
    half = x_f32.shape[-1] // 2
    return lax.bitcast_convert_type(
        bf[..., :half] | (bf[..., half:] << 16), jnp.int32)


def _from_i32_pairs(x_i32):
    u = lax.bitcast_convert_type(x_i32, jnp.uint32)
    lo = lax.bitcast_convert_type(u << 16, jnp.float32)
    hi = lax.bitcast_convert_type(u & jnp.uint32(0xFFFF0000), jnp.float32)
    return jnp.concatenate([lo, hi], axis=-1)


def kernel(bbox, x_emb, y_emb, gamma, beta):
    b, s, _ = bbox.shape
    offs = jnp.array([0, x_emb.shape[0], 0, x_emb.shape[0]], jnp.int32)
    idx = (bbox.reshape(b * s, 4) + offs).reshape(NWORK * NCHUNK, 4 * C)
    table = _as_i32_pairs(jnp.concatenate([x_emb, y_emb], axis=0))
    out = _sc_call(idx, table, _as_i32_pairs(gamma), _as_i32_pairs(beta))
    return out.reshape(b, s, D)


# speedup vs baseline: 1.9990x; 1.9990x over previous
import functools

import jax
import jax.numpy as jnp
from jax import lax
from jax.experimental import pallas as pl
from jax.experimental.pallas import tpu as pltpu
from jax.experimental.pallas import tpu_sc as plsc

D = 1024
D2 = D // 2
NTOK = 8192
NWORK = 32
TPW = NTOK // NWORK
C = 8
NCHUNK = TPW // C
NU = D2 // 16
UNROLL = 8
EPS = 1e-12
_PACK = plsc.PackFormat.INTERLEAVED


def _rsqrt16(v):
    i = lax.bitcast_convert_type(v, jnp.int32)
    i = jnp.int32(0x5F3759DF) - (i >> 1)
    y = lax.bitcast_convert_type(i, jnp.float32)
    for _ in range(3):
        y = y * (1.5 - 0.5 * v * y * y)
    return y


def _lane_total(v):
    lanes = lax.iota(jnp.int32, 16)
    for k in (8, 4, 2, 1):
        v = v + v.at[lanes ^ k].get(mode="promise_in_bounds")
    return v


def _bf(x):
    return plsc.bitcast(x, jnp.bfloat16)


def _i32(x):
    return plsc.bitcast(x, jnp.int32)


def _sc_body(idx_hbm, table_hbm, gamma_hbm, beta_hbm, out_hbm,
             idx_v, rows_v, buf_v, obuf_v, gam_v, bet_v,
             gsem0, gsem1, osem0, osem1):
    wid = lax.axis_index("s") * 2 + lax.axis_index("c")
    tok0 = wid * TPW

    pltpu.sync_copy(gamma_hbm, gam_v)
    pltpu.sync_copy(beta_hbm, bet_v)
    pltpu.sync_copy(idx_hbm.at[pl.ds(wid * NCHUNK, NCHUNK)], idx_v)

    def issue_gather(g, rows, gsem):
        return pltpu.async_copy(table_hbm.at[idx_v.at[g]], rows, gsem)

    def wait_gather(rows, gsem):
        pltpu.make_async_copy(table_hbm.at[idx_v.at[0]], rows, gsem).wait()

    def compute_chunk(g, rows, buf, obuf, osem):
        base_tok = tok0 + g * C

        def d_sum(d8, carry):
            s, q = carry
            ns, nq = [], []
            for t in range(C):
                st, qt = s[t], q[t]
                for j in range(UNROLL):
                    off = pl.ds(d8 * (16 * UNROLL) + j * 16, 16)
                    v = ((_bf(rows[4 * t + 0, off]) + _bf(rows[4 * t + 1, off]))
                         + (_bf(rows[4 * t + 2, off]) + _bf(rows[4 * t + 3, off])))
                    buf[t, off] = _i32(v)
                    a, b = plsc.unpack(v, format=_PACK)
                    st = st + (a + b)
                    qt = qt + (a * a + b * b)
                ns.append(st)
                nq.append(qt)
            return (tuple(ns), tuple(nq))

        zeros = jnp.zeros((16,), jnp.float32)
        s, q = lax.fori_loop(0, NU // UNROLL, d_sum,
                             ((zeros,) * C, (zeros,) * C))

        mvecs, rstds = [], []
        for t in range(C):
            mvec = _lane_total(s[t]) * (1.0 / D)
            var = _lane_total(q[t]) * (1.0 / D) - mvec * mvec
            rstd = _rsqrt16(var + EPS)
            mvecs.append(plsc.pack(mvec, mvec, format=_PACK))
            rstds.append(plsc.pack(rstd, rstd, format=_PACK))

        def d_norm(d8, carry):
            for j in range(UNROLL):
                o = d8 * (16 * UNROLL) + j * 16
                off = pl.ds(o, 16)
                gv = _bf(gam_v[off])
                bv = _bf(bet_v[off])
                for t in range(C):
                    v = _bf(buf[t, off])
                    y = (v - mvecs[t]) * rstds[t] * gv + bv
                    ylo, yhi = plsc.unpack(y, format=_PACK)
                    obuf[t, off] = ylo
                    obuf[t, pl.ds(D2 + o, 16)] = yhi
            return carry

        lax.fori_loop(0, NU // UNROLL, d_norm, 0)
        return pltpu.async_copy(obuf, out_hbm.at[pl.ds(base_tok, C)], osem)

    def wait_out(obuf, osem):
        pltpu.make_async_copy(obuf, out_hbm.at[pl.ds(tok0, C)], osem).wait()

    issue_gather(0, rows_v.at[0], gsem0)

    def pipe_body(g2, _):
        g = 2 * g2
        issue_gather(g + 1, rows_v.at[1], gsem1)
        wait_gather(rows_v.at[0], gsem0)

        @pl.when(g2 >= 1)
        def _():
            wait_out(obuf_v.at[0], osem0)

        compute_chunk(g, rows_v.at[0], buf_v.at[0], obuf_v.at[0], osem0)

        @pl.when(g + 2 < NCHUNK)
        def _():
            issue_gather(g + 2, rows_v.at[0], gsem0)

        wait_gather(rows_v.at[1], gsem1)

        @pl.when(g2 >= 1)
        def _():
            wait_out(obuf_v.at[1], osem1)

        compute_chunk(g + 1, rows_v.at[1], buf_v.at[1], obuf_v.at[1], osem1)
        return 0

    lax.fori_loop(0, NCHUNK // 2, pipe_body, 0)
    wait_out(obuf_v.at[0], osem0)
    wait_out(obuf_v.at[1], osem1)


@jax.jit
def _sc_call(idx, table, gamma, beta):
    mesh = plsc.VectorSubcoreMesh(core_axis_name="c", subcore_axis_name="s")
    kfn = functools.partial(
        pl.kernel, mesh=mesh,
        compiler_params=pltpu.CompilerParams(needs_layout_passes=False),
        out_type=jax.ShapeDtypeStruct((NTOK, D), jnp.float32),
        scratch_types=[
            pltpu.VMEM((NCHUNK, 4 * C), jnp.int32),
            pltpu.VMEM((2, 4 * C, D2), jnp.int32),
            pltpu.VMEM((2, C, D2), jnp.int32),
            pltpu.VMEM((2, C, D), jnp.float32),
            pltpu.VMEM((D2,), jnp.int32),
            pltpu.VMEM((D2,), jnp.int32),
            pltpu.SemaphoreType.DMA,
            pltpu.SemaphoreType.DMA,
            pltpu.SemaphoreType.DMA,
            pltpu.SemaphoreType.DMA,
        ],
    )(_sc_body)
    return kfn(idx, table, gamma, beta)


def _as_i32_pairs(x_f32):
    bits = lax.bitcast_convert_type(x_f32, jnp.uint32)
    r = bits + jnp.uint32(0x7FFF) + ((bits >> 16) & jnp.uint32(1))
    bf = r >> 16
    half = x_f32.shape[-1] // 2
    return lax.bitcast_convert_type(
        bf[..., :half] | (bf[..., half:] << 16), jnp.int32)


def kernel(bbox, x_emb, y_emb, gamma, beta):
    b, s, _ = bbox.shape
    offs = jnp.array([0, x_emb.shape[0], 0, x_emb.shape[0]], jnp.int32)
    idx = (bbox.reshape(b * s, 4) + offs).reshape(NWORK * NCHUNK, 4 * C)
    table = _as_i32_pairs(jnp.concatenate([x_emb, y_emb], axis=0))
    out = _sc_call(idx, table, _as_i32_pairs(gamma), _as_i32_pairs(beta))
    return out.reshape(b, s, D)
